# baseline (speedup 1.0000x reference)
import functools

import jax
import jax.numpy as jnp
from jax import lax
from jax.experimental import pallas as pl
from jax.experimental.pallas import tpu as pltpu
from jax.experimental.pallas import tpu_sc as plsc

_D = 64
_B = 16384
_NC = 2
_NS = 16
_NW = _NC * _NS
_BPW = _B // _NW
_L = 16

_mesh = plsc.VectorSubcoreMesh(core_axis_name="c", subcore_axis_name="s")


@functools.partial(
    pl.kernel,
    mesh=_mesh,
    out_type=[jax.ShapeDtypeStruct((_B, _D), jnp.float32)] * 3,
    scratch_types=[
        pltpu.VMEM((_BPW,), jnp.int32),
        pltpu.VMEM((_BPW,), jnp.int32),
        pltpu.VMEM((_BPW,), jnp.int32),
        pltpu.SemaphoreType.DMA,
    ],
)
def _gather3(h_idx, r_idx, t_idx, node_embs, rel_embs,
             h_out, r_out, t_out,
             h_ix, r_ix, t_ix, sem):
    wid = lax.axis_index("s") * _NC + lax.axis_index("c")
    base = wid * _BPW
    pltpu.sync_copy(h_idx.at[pl.ds(base, _BPW)], h_ix)
    pltpu.sync_copy(r_idx.at[pl.ds(base, _BPW)], r_ix)
    pltpu.sync_copy(t_idx.at[pl.ds(base, _BPW)], t_ix)

    def issue(q, carry):
        b16 = q * _L
        hv = h_ix[pl.ds(b16, _L)]
        rv = r_ix[pl.ds(b16, _L)]
        tv = t_ix[pl.ds(b16, _L)]
        for c in range(_L):
            dst = pl.ds(base + b16 + c, 1)
            pltpu.async_copy(node_embs.at[pl.ds(hv[c], 1)], h_out.at[dst], sem)
            pltpu.async_copy(rel_embs.at[pl.ds(rv[c], 1)], r_out.at[dst], sem)
            pltpu.async_copy(node_embs.at[pl.ds(tv[c], 1)], t_out.at[dst], sem)
        return carry

    lax.fori_loop(0, _BPW // _L, issue, 0)

    def drain(i, carry):
        pltpu.make_async_copy(node_embs.at[pl.ds(0, 1)], h_out.at[pl.ds(base + i, 1)], sem).wait()
        pltpu.make_async_copy(rel_embs.at[pl.ds(0, 1)], r_out.at[pl.ds(base + i, 1)], sem).wait()
        pltpu.make_async_copy(node_embs.at[pl.ds(0, 1)], t_out.at[pl.ds(base + i, 1)], sem).wait()
        return carry

    lax.fori_loop(0, _BPW, drain, 0)


def kernel(triplets, node_embs, rel_embs):
    tri = triplets.astype(jnp.int32)
    h_idx = tri[:, 0].reshape(_B)
    r_idx = tri[:, 1].reshape(_B)
    t_idx = tri[:, 2].reshape(_B)
    head, rel, tail = _gather3(h_idx, r_idx, t_idx, node_embs, rel_embs)
    return (head, rel, tail)


# speedup vs baseline: 2.0119x; 2.0119x over previous
import functools

import jax
import jax.numpy as jnp
from jax import lax
from jax.experimental import pallas as pl
from jax.experimental.pallas import tpu as pltpu
from jax.experimental.pallas import tpu_sc as plsc

_D = 64
_B = 16384
_NC = 2
_NS = 16
_NW = _NC * _NS
_BPW = _B // _NW
_L = 16
_CH = 128
_NCHK = _BPW // _CH
_NBUF = 2

_mesh = plsc.VectorSubcoreMesh(core_axis_name="c", subcore_axis_name="s")


@functools.partial(
    pl.kernel,
    mesh=_mesh,
    out_type=[jax.ShapeDtypeStruct((_B, _D), jnp.float32)] * 3,
    scratch_types=[
        pltpu.VMEM((_BPW,), jnp.int32),
        pltpu.VMEM((_BPW,), jnp.int32),
        pltpu.VMEM((_BPW,), jnp.int32),
        pltpu.VMEM((_NBUF, _CH, _D), jnp.float32),
        pltpu.VMEM((_NBUF, _CH, _D), jnp.float32),
        pltpu.VMEM((_NBUF, _CH, _D), jnp.float32),
        pltpu.SemaphoreType.DMA,
        pltpu.SemaphoreType.DMA,
    ],
)
def _gather3(h_idx, r_idx, t_idx, node_embs, rel_embs,
             h_out, r_out, t_out,
             h_ix, r_ix, t_ix, h_buf, r_buf, t_buf, sem0, sem1):
    sems = (sem0, sem1)
    wid = lax.axis_index("s") * _NC + lax.axis_index("c")
    base = wid * _BPW
    pltpu.sync_copy(h_idx.at[pl.ds(base, _BPW)], h_ix)
    pltpu.sync_copy(r_idx.at[pl.ds(base, _BPW)], r_ix)
    pltpu.sync_copy(t_idx.at[pl.ds(base, _BPW)], t_ix)

    def issue_chunk(c, slot):
        sem = sems[slot]
        def issue16(q, carry):
            b16 = c * _CH + q * _L
            hv = h_ix[pl.ds(b16, _L)]
            rv = r_ix[pl.ds(b16, _L)]
            tv = t_ix[pl.ds(b16, _L)]
            for k in range(_L):
                dst = (slot, pl.ds(q * _L + k, 1))
                pltpu.async_copy(node_embs.at[pl.ds(hv[k], 1)], h_buf.at[dst], sem)
                pltpu.async_copy(rel_embs.at[pl.ds(rv[k], 1)], r_buf.at[dst], sem)
                pltpu.async_copy(node_embs.at[pl.ds(tv[k], 1)], t_buf.at[dst], sem)
            return carry
        lax.fori_loop(0, _CH // _L, issue16, 0)

    def drain_chunk(slot):
        sem = sems[slot]
        def drain1(i, carry):
            pltpu.make_async_copy(node_embs.at[pl.ds(0, 1)], h_buf.at[slot, pl.ds(i, 1)], sem).wait()
            pltpu.make_async_copy(rel_embs.at[pl.ds(0, 1)], r_buf.at[slot, pl.ds(i, 1)], sem).wait()
            pltpu.make_async_copy(node_embs.at[pl.ds(0, 1)], t_buf.at[slot, pl.ds(i, 1)], sem).wait()
            return carry
        lax.fori_loop(0, _CH, drain1, 0)

    def write_chunk(c, slot):
        dst = pl.ds(base + c * _CH, _CH)
        pltpu.sync_copy(h_buf.at[slot], h_out.at[dst])
        pltpu.sync_copy(r_buf.at[slot], r_out.at[dst])
        pltpu.sync_copy(t_buf.at[slot], t_out.at[dst])

    issue_chunk(0, 0)
    for c in range(_NCHK):
        if c + 1 < _NCHK:
            issue_chunk(c + 1, (c + 1) % _NBUF)
        drain_chunk(c % _NBUF)
        write_chunk(c, c % _NBUF)


def kernel(triplets, node_embs, rel_embs):
    tri = triplets.astype(jnp.int32)
    h_idx = tri[:, 0].reshape(_B)
    r_idx = tri[:, 1].reshape(_B)
    t_idx = tri[:, 2].reshape(_B)
    head, rel, tail = _gather3(h_idx, r_idx, t_idx, node_embs, rel_embs)
    return (head, rel, tail)
